# Initial kernel scaffold; baseline (speedup 1.0000x reference)
import jax
import jax.numpy as jnp
from jax.experimental import pallas as pl


def kernel(node_ids, edges, edge_types, embed_table, edge_embed_table, ggnn_weight, w_ih, w_hh, b_ih, b_hh, gate_w, gate_b):
    raise NotImplementedError("write your pallas kernel here")


# speedup vs baseline: 9.7169x; 9.7169x over previous
import functools

import jax
import jax.numpy as jnp
from jax import lax
from jax.experimental import pallas as pl
from jax.experimental.pallas import tpu as pltpu
from jax.experimental.pallas import tpu_sc as plsc

N = 10000
NP = 10240
H = 128
E = 320000
EP = 327680
L = 4
NC = 2
NS = 16
NW = NC * NS
EW = EP // NW
NWIN = 80
WSZ = EW // NWIN
IDW = NP // NW
BN = 1024
_HI = lax.Precision.HIGHEST


@functools.cache
def _sc_embed_kernel():
    mesh = plsc.VectorSubcoreMesh(
        core_axis_name="c", subcore_axis_name="s",
        num_cores=NC, num_subcores=NS)
    return pl.kernel(
        _sc_embed_body,
        out_type=[jax.ShapeDtypeStruct((NP, H), jnp.float32),
                  jax.ShapeDtypeStruct((NW, EW), jnp.int32)],
        mesh=mesh,
        scratch_types=[
            pltpu.VMEM((IDW,), jnp.int32),
            pltpu.VMEM((IDW, H), jnp.float32),
            pltpu.VMEM((EW,), jnp.int32),
            pltpu.VMEM((EW,), jnp.int32),
            pltpu.SemaphoreType.DMA,
        ],
    )


def _sc_embed_body(emb_hbm, ids_hbm, src_hbm, et_hbm, x_out, s2_out,
                   idx_v, rows_v, sv, ev, sem):
    c = lax.axis_index("c")
    s = lax.axis_index("s")
    wid = c * NS + s
    pltpu.sync_copy(ids_hbm.at[wid], idx_v)

    def _inc(i, carry):
        sl = pl.ds(i * 16, 16)
        idx_v[sl] = idx_v[sl] + 1
        return carry
    lax.fori_loop(0, IDW // 16, _inc, 0)

    cps = []
    for j in range(IDW // 80):
        cps.append(pltpu.async_copy(
            emb_hbm.at[idx_v.at[pl.ds(j * 80, 80)]],
            rows_v.at[pl.ds(j * 80, 80)], sem))
    for cp in cps:
        cp.wait()
    pltpu.sync_copy(rows_v, x_out.at[pl.ds(wid * IDW, IDW)])
    pltpu.sync_copy(src_hbm.at[wid], sv)
    pltpu.sync_copy(et_hbm.at[wid], ev)

    def _s2(i, carry):
        sl = pl.ds(i * 16, 16)
        sv[sl] = (ev[sl] - 1) * NP + sv[sl]
        return carry
    lax.fori_loop(0, EW // 16, _s2, 0)
    pltpu.sync_copy(sv, s2_out.at[wid])


@functools.cache
def _sc_scatter_kernel():
    mesh = plsc.VectorSubcoreMesh(
        core_axis_name="c", subcore_axis_name="s",
        num_cores=NC, num_subcores=NS)
    return pl.kernel(
        _sc_scatter_body,
        out_type=jax.ShapeDtypeStruct((NC, NP, H), jnp.float32),
        mesh=mesh,
        scratch_types=[
            pltpu.VMEM((2, WSZ), jnp.int32),
            pltpu.VMEM((2, WSZ), jnp.int32),
            pltpu.VMEM((2, WSZ, H), jnp.float32),
            pltpu.VMEM_SHARED((NP, H), jnp.float32),
            pltpu.SemaphoreType.DMA,
            pltpu.SemaphoreType.DMA,
            pltpu.SemaphoreType.DMA,
            pltpu.SemaphoreType.DMA,
        ],
    )


def _sc_scatter_body(m7_hbm, s2_hbm, dst_hbm, z_hbm, agg_out,
                     s2b, db, rows, agg_sh, isem0, isem1, gsem0, gsem1):
    c = lax.axis_index("c")
    s = lax.axis_index("s")
    wid = c * NS + s
    rows_per_tile = NP // NS
    pltpu.sync_copy(z_hbm, agg_sh.at[pl.ds(s * rows_per_tile, rows_per_tile)])
    plsc.subcore_barrier()
    isems = (isem0, isem1)
    gsems = (gsem0, gsem1)
    pltpu.async_copy(s2_hbm.at[wid, 0], s2b.at[0], isem0)
    pltpu.async_copy(dst_hbm.at[wid, 0], db.at[0], isem0)
    pltpu.async_copy(s2_hbm.at[wid, 1], s2b.at[1], isem1)
    pltpu.async_copy(dst_hbm.at[wid, 1], db.at[1], isem1)
    pltpu.make_async_copy(s2_hbm.at[wid, 0], s2b.at[0], isem0).wait()
    pltpu.make_async_copy(dst_hbm.at[wid, 0], db.at[0], isem0).wait()
    pltpu.async_copy(m7_hbm.at[s2b.at[0]], rows.at[0], gsem0)

    def _win(i, carry):
        for b in range(2):
            w = i * 2 + b
            nb = 1 - b

            @pl.when(w + 1 < NWIN)
            def _():
                pltpu.make_async_copy(
                    s2_hbm.at[wid, 0], s2b.at[nb], isems[nb]).wait()
                pltpu.make_async_copy(
                    dst_hbm.at[wid, 0], db.at[nb], isems[nb]).wait()
            pltpu.make_async_copy(
                m7_hbm.at[s2b.at[b]], rows.at[b], gsems[b]).wait()

            @pl.when(w + 1 < NWIN)
            def _():
                pltpu.async_copy(m7_hbm.at[s2b.at[nb]], rows.at[nb],
                                 gsems[nb])
            pltpu.sync_copy(rows.at[b], agg_sh.at[db.at[b]], add=True)

            @pl.when(w + 2 < NWIN)
            def _():
                pltpu.async_copy(s2_hbm.at[wid, w + 2], s2b.at[b], isems[b])
                pltpu.async_copy(dst_hbm.at[wid, w + 2], db.at[b], isems[b])
        return carry
    lax.fori_loop(0, NWIN // 2, _win, 0)
    plsc.subcore_barrier()
    pltpu.sync_copy(agg_sh.at[pl.ds(s * rows_per_tile, rows_per_tile)],
                    agg_out.at[c, pl.ds(s * rows_per_tile, rows_per_tile)])


def _scaled_mm(ee, x, w):
    ew7 = jnp.mean(ee, axis=1)
    m = lax.dot_general(x, w, (((1,), (0,)), ((), ())),
                        preferred_element_type=jnp.float32, precision=_HI)
    return ew7[:, None, None] * m[None]


def _m7_body(ee_ref, w_ref, x_ref, out_ref):
    out_ref[...] = _scaled_mm(ee_ref[...], x_ref[...], w_ref[...])


def _gru(agg, x, wih, whh, bih, bhh):
    gi = lax.dot_general(agg, wih, (((1,), (1,)), ((), ())),
                         preferred_element_type=jnp.float32,
                         precision=_HI) + bih
    gh = lax.dot_general(x, whh, (((1,), (1,)), ((), ())),
                         preferred_element_type=jnp.float32,
                         precision=_HI) + bhh
    r = jax.nn.sigmoid(gi[:, :H] + gh[:, :H])
    z = jax.nn.sigmoid(gi[:, H:2 * H] + gh[:, H:2 * H])
    n = jnp.tanh(gi[:, 2 * H:] + r * gh[:, 2 * H:])
    return (1.0 - z) * n + z * x


def _ga_body(wih_ref, whh_ref, bih_ref, bhh_ref, ee_ref, wn_ref,
             agg_ref, x_ref, xo_ref, m7_ref):
    xn = _gru(agg_ref[0] + agg_ref[1], x_ref[...], wih_ref[...], whh_ref[...],
              bih_ref[...], bhh_ref[...])
    xo_ref[...] = xn
    m7_ref[...] = _scaled_mm(ee_ref[...], xn, wn_ref[...])


def _gru_body(wih_ref, whh_ref, bih_ref, bhh_ref, agg_ref, x_ref, xo_ref):
    xo_ref[...] = _gru(agg_ref[0] + agg_ref[1], x_ref[...], wih_ref[...],
                       whh_ref[...], bih_ref[...], bhh_ref[...])


def _pool_body(x_ref, gw_ref, gb_ref, o_ref):
    x = x_ref[...]
    raw = jnp.sum(x * gw_ref[...], axis=1, keepdims=True) + gb_ref[0, 0]
    g = jax.nn.sigmoid(raw)
    valid = lax.broadcasted_iota(jnp.int32, (NP, 1), 0) < N
    g = jnp.where(valid, g, 0.0)
    a = jnp.where(valid, jnp.exp(g - jnp.max(g)), 0.0)
    a = a / jnp.sum(a)
    o_ref[...] = lax.dot_general(a, x, (((0,), (0,)), ((), ())),
                                 preferred_element_type=jnp.float32,
                                 precision=_HI)


_full = pl.BlockSpec(index_map=lambda i: (0,) * 2)
_full3 = pl.BlockSpec(index_map=lambda i: (0,) * 3)


def _m7_call(ee, w, x):
    return pl.pallas_call(
        _m7_body,
        grid=(NP // BN,),
        in_specs=[pl.BlockSpec((7, H), lambda i: (0, 0)),
                  pl.BlockSpec((H, H), lambda i: (0, 0)),
                  pl.BlockSpec((BN, H), lambda i: (i, 0))],
        out_specs=pl.BlockSpec((7, BN, H), lambda i: (0, i, 0)),
        out_shape=jax.ShapeDtypeStruct((7, NP, H), jnp.float32),
    )(ee, w, x)


def _ga_call(wih, whh, bih, bhh, ee, wn, agg2, x):
    return pl.pallas_call(
        _ga_body,
        grid=(NP // BN,),
        in_specs=[pl.BlockSpec((3 * H, H), lambda i: (0, 0)),
                  pl.BlockSpec((3 * H, H), lambda i: (0, 0)),
                  pl.BlockSpec((1, 3 * H), lambda i: (0, 0)),
                  pl.BlockSpec((1, 3 * H), lambda i: (0, 0)),
                  pl.BlockSpec((7, H), lambda i: (0, 0)),
                  pl.BlockSpec((H, H), lambda i: (0, 0)),
                  pl.BlockSpec((NC, BN, H), lambda i: (0, i, 0)),
                  pl.BlockSpec((BN, H), lambda i: (i, 0))],
        out_specs=[pl.BlockSpec((BN, H), lambda i: (i, 0)),
                   pl.BlockSpec((7, BN, H), lambda i: (0, i, 0))],
        out_shape=[jax.ShapeDtypeStruct((NP, H), jnp.float32),
                   jax.ShapeDtypeStruct((7, NP, H), jnp.float32)],
    )(wih, whh, bih, bhh, ee, wn, agg2, x)


def _gru_call(wih, whh, bih, bhh, agg2, x):
    return pl.pallas_call(
        _gru_body,
        grid=(NP // BN,),
        in_specs=[pl.BlockSpec((3 * H, H), lambda i: (0, 0)),
                  pl.BlockSpec((3 * H, H), lambda i: (0, 0)),
                  pl.BlockSpec((1, 3 * H), lambda i: (0, 0)),
                  pl.BlockSpec((1, 3 * H), lambda i: (0, 0)),
                  pl.BlockSpec((NC, BN, H), lambda i: (0, i, 0)),
                  pl.BlockSpec((BN, H), lambda i: (i, 0))],
        out_specs=pl.BlockSpec((BN, H), lambda i: (i, 0)),
        out_shape=jax.ShapeDtypeStruct((NP, H), jnp.float32),
    )(wih, whh, bih, bhh, agg2, x)


def _pool_call(x, gw, gb):
    return pl.pallas_call(
        _pool_body,
        in_specs=[pl.BlockSpec((NP, H), lambda: (0, 0)),
                  pl.BlockSpec((1, H), lambda: (0, 0)),
                  pl.BlockSpec((1, 1), lambda: (0, 0))],
        out_specs=pl.BlockSpec((1, H), lambda: (0, 0)),
        out_shape=jax.ShapeDtypeStruct((1, H), jnp.float32),
    )(x, gw, gb)


def kernel(node_ids, edges, edge_types, embed_table, edge_embed_table,
           ggnn_weight, w_ih, w_hh, b_ih, b_hh, gate_w, gate_b):
    src = edges[0].astype(jnp.int32)
    dst = edges[1].astype(jnp.int32)
    et = edge_types.astype(jnp.int32)
    padi = jnp.arange(EP - E, dtype=jnp.int32) % 16
    src_p = jnp.concatenate([src, padi]).reshape(NW, EW)
    et_p = jnp.concatenate([et, jnp.ones((EP - E,), jnp.int32)]).reshape(NW, EW)
    dst_p = jnp.concatenate([dst, N + padi]).reshape(NW, NWIN, WSZ)
    ids_p = jnp.concatenate(
        [node_ids.astype(jnp.int32),
         jnp.arange(NP - N, dtype=jnp.int32) % 64]).reshape(NW, IDW)
    zrows = jnp.zeros((NP // NS, H), jnp.float32)
    bih = b_ih.reshape(1, 3 * H)
    bhh = b_hh.reshape(1, 3 * H)
    gb = gate_b.reshape(1, 1)

    x, s2 = _sc_embed_kernel()(embed_table, ids_p, src_p, et_p)
    s2 = s2.reshape(NW, NWIN, WSZ)
    m7 = _m7_call(edge_embed_table, ggnn_weight[0], x).reshape(7 * NP, H)
    for layer in range(L):
        agg2 = _sc_scatter_kernel()(m7, s2, dst_p, zrows)
        if layer + 1 < L:
            x, m7 = _ga_call(w_ih, w_hh, bih, bhh, edge_embed_table,
                             ggnn_weight[layer + 1], agg2, x)
            m7 = m7.reshape(7 * NP, H)
        else:
            x = _gru_call(w_ih, w_hh, bih, bhh, agg2, x)
    return _pool_call(x, gate_w, gb)


# speedup vs baseline: 11.0258x; 1.1347x over previous
import functools

import jax
import jax.numpy as jnp
from jax import lax
from jax.experimental import pallas as pl
from jax.experimental.pallas import tpu as pltpu
from jax.experimental.pallas import tpu_sc as plsc

N = 10000
NP = 10240
H = 128
E = 320000
EP = 327680
L = 4
NC = 2
NS = 16
NW = NC * NS
EW = EP // NW
NWIN = 80
WSZ = EW // NWIN
IDW = NP // NW
BN = 1024
_HI = lax.Precision.DEFAULT


@functools.cache
def _sc_embed_kernel():
    mesh = plsc.VectorSubcoreMesh(
        core_axis_name="c", subcore_axis_name="s",
        num_cores=NC, num_subcores=NS)
    return pl.kernel(
        _sc_embed_body,
        out_type=[jax.ShapeDtypeStruct((NP, H), jnp.float32),
                  jax.ShapeDtypeStruct((NW, EW), jnp.int32)],
        mesh=mesh,
        scratch_types=[
            pltpu.VMEM((IDW,), jnp.int32),
            pltpu.VMEM((IDW, H), jnp.float32),
            pltpu.VMEM((EW,), jnp.int32),
            pltpu.VMEM((EW,), jnp.int32),
            pltpu.SemaphoreType.DMA,
        ],
    )


def _sc_embed_body(emb_hbm, ids_hbm, src_hbm, et_hbm, x_out, s2_out,
                   idx_v, rows_v, sv, ev, sem):
    c = lax.axis_index("c")
    s = lax.axis_index("s")
    wid = c * NS + s
    pltpu.sync_copy(ids_hbm.at[wid], idx_v)

    def _inc(i, carry):
        sl = pl.ds(i * 16, 16)
        idx_v[sl] = idx_v[sl] + 1
        return carry
    lax.fori_loop(0, IDW // 16, _inc, 0)

    cps = []
    for j in range(IDW // 80):
        cps.append(pltpu.async_copy(
            emb_hbm.at[idx_v.at[pl.ds(j * 80, 80)]],
            rows_v.at[pl.ds(j * 80, 80)], sem))
    for cp in cps:
        cp.wait()
    pltpu.sync_copy(rows_v, x_out.at[pl.ds(wid * IDW, IDW)])
    pltpu.sync_copy(src_hbm.at[wid], sv)
    pltpu.sync_copy(et_hbm.at[wid], ev)

    def _s2(i, carry):
        sl = pl.ds(i * 16, 16)
        sv[sl] = (ev[sl] - 1) * NP + sv[sl]
        return carry
    lax.fori_loop(0, EW // 16, _s2, 0)
    pltpu.sync_copy(sv, s2_out.at[wid])


@functools.cache
def _sc_scatter_kernel():
    mesh = plsc.VectorSubcoreMesh(
        core_axis_name="c", subcore_axis_name="s",
        num_cores=NC, num_subcores=NS)
    return pl.kernel(
        _sc_scatter_body,
        out_type=jax.ShapeDtypeStruct((NC, NP, H), jnp.float32),
        mesh=mesh,
        scratch_types=[
            pltpu.VMEM((2, WSZ), jnp.int32),
            pltpu.VMEM((2, WSZ), jnp.int32),
            pltpu.VMEM((2, WSZ, H), jnp.float32),
            pltpu.VMEM_SHARED((NP, H), jnp.float32),
            pltpu.SemaphoreType.DMA,
            pltpu.SemaphoreType.DMA,
            pltpu.SemaphoreType.DMA,
            pltpu.SemaphoreType.DMA,
        ],
    )


def _sc_scatter_body(m7_hbm, s2_hbm, dst_hbm, z_hbm, agg_out,
                     s2b, db, rows, agg_sh, isem0, isem1, gsem0, gsem1):
    c = lax.axis_index("c")
    s = lax.axis_index("s")
    wid = c * NS + s
    rows_per_tile = NP // NS
    pltpu.sync_copy(z_hbm, agg_sh.at[pl.ds(s * rows_per_tile, rows_per_tile)])
    plsc.subcore_barrier()
    isems = (isem0, isem1)
    gsems = (gsem0, gsem1)
    pltpu.async_copy(s2_hbm.at[wid, 0], s2b.at[0], isem0)
    pltpu.async_copy(dst_hbm.at[wid, 0], db.at[0], isem0)
    pltpu.async_copy(s2_hbm.at[wid, 1], s2b.at[1], isem1)
    pltpu.async_copy(dst_hbm.at[wid, 1], db.at[1], isem1)
    pltpu.make_async_copy(s2_hbm.at[wid, 0], s2b.at[0], isem0).wait()
    pltpu.make_async_copy(dst_hbm.at[wid, 0], db.at[0], isem0).wait()
    pltpu.async_copy(m7_hbm.at[s2b.at[0]], rows.at[0], gsem0)

    def _win(i, carry):
        for b in range(2):
            w = i * 2 + b
            nb = 1 - b

            @pl.when(w + 1 < NWIN)
            def _():
                pltpu.make_async_copy(
                    s2_hbm.at[wid, 0], s2b.at[nb], isems[nb]).wait()
                pltpu.make_async_copy(
                    dst_hbm.at[wid, 0], db.at[nb], isems[nb]).wait()
            pltpu.make_async_copy(
                m7_hbm.at[s2b.at[b]], rows.at[b], gsems[b]).wait()

            @pl.when(w + 1 < NWIN)
            def _():
                pltpu.async_copy(m7_hbm.at[s2b.at[nb]], rows.at[nb],
                                 gsems[nb])
            pltpu.sync_copy(rows.at[b], agg_sh.at[db.at[b]], add=True)

            @pl.when(w + 2 < NWIN)
            def _():
                pltpu.async_copy(s2_hbm.at[wid, w + 2], s2b.at[b], isems[b])
                pltpu.async_copy(dst_hbm.at[wid, w + 2], db.at[b], isems[b])
        return carry
    lax.fori_loop(0, NWIN // 2, _win, 0)
    plsc.subcore_barrier()
    pltpu.sync_copy(agg_sh.at[pl.ds(s * rows_per_tile, rows_per_tile)],
                    agg_out.at[c, pl.ds(s * rows_per_tile, rows_per_tile)])


def _scaled_mm(ee, x, w):
    ew7 = jnp.mean(ee, axis=1)
    m = lax.dot_general(x, w, (((1,), (0,)), ((), ())),
                        preferred_element_type=jnp.float32, precision=_HI)
    return ew7[:, None, None] * m[None]


def _m7_body(ee_ref, w_ref, x_ref, out_ref):
    out_ref[...] = _scaled_mm(ee_ref[...], x_ref[...], w_ref[...])


def _gru(agg, x, wih, whh, bih, bhh):
    gi = lax.dot_general(agg, wih, (((1,), (1,)), ((), ())),
                         preferred_element_type=jnp.float32,
                         precision=_HI) + bih
    gh = lax.dot_general(x, whh, (((1,), (1,)), ((), ())),
                         preferred_element_type=jnp.float32,
                         precision=_HI) + bhh
    r = jax.nn.sigmoid(gi[:, :H] + gh[:, :H])
    z = jax.nn.sigmoid(gi[:, H:2 * H] + gh[:, H:2 * H])
    n = jnp.tanh(gi[:, 2 * H:] + r * gh[:, 2 * H:])
    return (1.0 - z) * n + z * x


def _ga_body(wih_ref, whh_ref, bih_ref, bhh_ref, ee_ref, wn_ref,
             agg_ref, x_ref, xo_ref, m7_ref):
    xn = _gru(agg_ref[0] + agg_ref[1], x_ref[...], wih_ref[...], whh_ref[...],
              bih_ref[...], bhh_ref[...])
    xo_ref[...] = xn
    m7_ref[...] = _scaled_mm(ee_ref[...], xn, wn_ref[...])


def _gru_body(wih_ref, whh_ref, bih_ref, bhh_ref, agg_ref, x_ref, xo_ref):
    xo_ref[...] = _gru(agg_ref[0] + agg_ref[1], x_ref[...], wih_ref[...],
                       whh_ref[...], bih_ref[...], bhh_ref[...])


def _pool_body(x_ref, gw_ref, gb_ref, o_ref):
    x = x_ref[...]
    raw = jnp.sum(x * gw_ref[...], axis=1, keepdims=True) + gb_ref[0, 0]
    g = jax.nn.sigmoid(raw)
    valid = lax.broadcasted_iota(jnp.int32, (NP, 1), 0) < N
    g = jnp.where(valid, g, 0.0)
    a = jnp.where(valid, jnp.exp(g - jnp.max(g)), 0.0)
    a = a / jnp.sum(a)
    o_ref[...] = lax.dot_general(a, x, (((0,), (0,)), ((), ())),
                                 preferred_element_type=jnp.float32,
                                 precision=_HI)


_full = pl.BlockSpec(index_map=lambda i: (0,) * 2)
_full3 = pl.BlockSpec(index_map=lambda i: (0,) * 3)


def _m7_call(ee, w, x):
    return pl.pallas_call(
        _m7_body,
        grid=(NP // BN,),
        in_specs=[pl.BlockSpec((7, H), lambda i: (0, 0)),
                  pl.BlockSpec((H, H), lambda i: (0, 0)),
                  pl.BlockSpec((BN, H), lambda i: (i, 0))],
        out_specs=pl.BlockSpec((7, BN, H), lambda i: (0, i, 0)),
        out_shape=jax.ShapeDtypeStruct((7, NP, H), jnp.float32),
    )(ee, w, x)


def _ga_call(wih, whh, bih, bhh, ee, wn, agg2, x):
    return pl.pallas_call(
        _ga_body,
        grid=(NP // BN,),
        in_specs=[pl.BlockSpec((3 * H, H), lambda i: (0, 0)),
                  pl.BlockSpec((3 * H, H), lambda i: (0, 0)),
                  pl.BlockSpec((1, 3 * H), lambda i: (0, 0)),
                  pl.BlockSpec((1, 3 * H), lambda i: (0, 0)),
                  pl.BlockSpec((7, H), lambda i: (0, 0)),
                  pl.BlockSpec((H, H), lambda i: (0, 0)),
                  pl.BlockSpec((NC, BN, H), lambda i: (0, i, 0)),
                  pl.BlockSpec((BN, H), lambda i: (i, 0))],
        out_specs=[pl.BlockSpec((BN, H), lambda i: (i, 0)),
                   pl.BlockSpec((7, BN, H), lambda i: (0, i, 0))],
        out_shape=[jax.ShapeDtypeStruct((NP, H), jnp.float32),
                   jax.ShapeDtypeStruct((7, NP, H), jnp.float32)],
    )(wih, whh, bih, bhh, ee, wn, agg2, x)


def _gru_call(wih, whh, bih, bhh, agg2, x):
    return pl.pallas_call(
        _gru_body,
        grid=(NP // BN,),
        in_specs=[pl.BlockSpec((3 * H, H), lambda i: (0, 0)),
                  pl.BlockSpec((3 * H, H), lambda i: (0, 0)),
                  pl.BlockSpec((1, 3 * H), lambda i: (0, 0)),
                  pl.BlockSpec((1, 3 * H), lambda i: (0, 0)),
                  pl.BlockSpec((NC, BN, H), lambda i: (0, i, 0)),
                  pl.BlockSpec((BN, H), lambda i: (i, 0))],
        out_specs=pl.BlockSpec((BN, H), lambda i: (i, 0)),
        out_shape=jax.ShapeDtypeStruct((NP, H), jnp.float32),
    )(wih, whh, bih, bhh, agg2, x)


def _pool_call(x, gw, gb):
    return pl.pallas_call(
        _pool_body,
        in_specs=[pl.BlockSpec((NP, H), lambda: (0, 0)),
                  pl.BlockSpec((1, H), lambda: (0, 0)),
                  pl.BlockSpec((1, 1), lambda: (0, 0))],
        out_specs=pl.BlockSpec((1, H), lambda: (0, 0)),
        out_shape=jax.ShapeDtypeStruct((1, H), jnp.float32),
    )(x, gw, gb)


def kernel(node_ids, edges, edge_types, embed_table, edge_embed_table,
           ggnn_weight, w_ih, w_hh, b_ih, b_hh, gate_w, gate_b):
    src = edges[0].astype(jnp.int32)
    dst = edges[1].astype(jnp.int32)
    et = edge_types.astype(jnp.int32)
    padi = jnp.arange(EP - E, dtype=jnp.int32) % 16
    src_p = jnp.concatenate([src, padi]).reshape(NW, EW)
    et_p = jnp.concatenate([et, jnp.ones((EP - E,), jnp.int32)]).reshape(NW, EW)
    dst_p = jnp.concatenate([dst, N + padi]).reshape(NW, NWIN, WSZ)
    ids_p = jnp.concatenate(
        [node_ids.astype(jnp.int32),
         jnp.arange(NP - N, dtype=jnp.int32) % 64]).reshape(NW, IDW)
    zrows = jnp.zeros((NP // NS, H), jnp.float32)
    bih = b_ih.reshape(1, 3 * H)
    bhh = b_hh.reshape(1, 3 * H)
    gb = gate_b.reshape(1, 1)

    x, s2 = _sc_embed_kernel()(embed_table, ids_p, src_p, et_p)
    s2 = s2.reshape(NW, NWIN, WSZ)
    m7 = _m7_call(edge_embed_table, ggnn_weight[0], x).reshape(7 * NP, H)
    for layer in range(L):
        agg2 = _sc_scatter_kernel()(m7, s2, dst_p, zrows)
        if layer + 1 < L:
            x, m7 = _ga_call(w_ih, w_hh, bih, bhh, edge_embed_table,
                             ggnn_weight[layer + 1], agg2, x)
            m7 = m7.reshape(7 * NP, H)
        else:
            x = _gru_call(w_ih, w_hh, bih, bhh, agg2, x)
    return _pool_call(x, gate_w, gb)


# speedup vs baseline: 11.0642x; 1.0035x over previous
import functools

import jax
import jax.numpy as jnp
from jax import lax
from jax.experimental import pallas as pl
from jax.experimental.pallas import tpu as pltpu
from jax.experimental.pallas import tpu_sc as plsc

N = 10000
NP = 10240
H = 128
E = 320000
EP = 327680
L = 4
NC = 2
NS = 16
NW = NC * NS
EW = EP // NW
NWIN = 80
WSZ = EW // NWIN
IDW = NP // NW
BN = 1024
_HI = lax.Precision.DEFAULT


@functools.cache
def _sc_embed_kernel():
    mesh = plsc.VectorSubcoreMesh(
        core_axis_name="c", subcore_axis_name="s",
        num_cores=NC, num_subcores=NS)
    return pl.kernel(
        _sc_embed_body,
        out_type=[jax.ShapeDtypeStruct((NP, H), jnp.float32),
                  jax.ShapeDtypeStruct((NW, EW), jnp.int32)],
        mesh=mesh,
        scratch_types=[
            pltpu.VMEM((IDW,), jnp.int32),
            pltpu.VMEM((IDW, H), jnp.float32),
            pltpu.VMEM((EW,), jnp.int32),
            pltpu.VMEM((EW,), jnp.int32),
            pltpu.SemaphoreType.DMA,
        ],
    )


def _sc_embed_body(emb_hbm, ids_hbm, src_hbm, et_hbm, x_out, s2_out,
                   idx_v, rows_v, sv, ev, sem):
    c = lax.axis_index("c")
    s = lax.axis_index("s")
    wid = c * NS + s
    pltpu.sync_copy(ids_hbm.at[wid], idx_v)

    def _inc(i, carry):
        sl = pl.ds(i * 16, 16)
        idx_v[sl] = idx_v[sl] + 1
        return carry
    lax.fori_loop(0, IDW // 16, _inc, 0)

    cps = []
    for j in range(IDW // 80):
        cps.append(pltpu.async_copy(
            emb_hbm.at[idx_v.at[pl.ds(j * 80, 80)]],
            rows_v.at[pl.ds(j * 80, 80)], sem))
    for cp in cps:
        cp.wait()
    pltpu.sync_copy(rows_v, x_out.at[pl.ds(wid * IDW, IDW)])
    pltpu.sync_copy(src_hbm.at[wid], sv)
    pltpu.sync_copy(et_hbm.at[wid], ev)

    def _s2(i, carry):
        sl = pl.ds(i * 16, 16)
        sv[sl] = (ev[sl] - 1) * NP + sv[sl]
        return carry
    lax.fori_loop(0, EW // 16, _s2, 0)
    pltpu.sync_copy(sv, s2_out.at[wid])


@functools.cache
def _sc_scatter_kernel():
    mesh = plsc.VectorSubcoreMesh(
        core_axis_name="c", subcore_axis_name="s",
        num_cores=NC, num_subcores=NS)
    return pl.kernel(
        _sc_scatter_body,
        out_type=jax.ShapeDtypeStruct((NC, NP, H), jnp.float32),
        mesh=mesh,
        scratch_types=[
            pltpu.VMEM((4, WSZ), jnp.int32),
            pltpu.VMEM((4, WSZ), jnp.int32),
            pltpu.VMEM((2, WSZ, H), jnp.float32),
            pltpu.VMEM_SHARED((NP, H), jnp.float32),
            pltpu.SemaphoreType.DMA,
            pltpu.SemaphoreType.DMA,
            pltpu.SemaphoreType.DMA,
            pltpu.SemaphoreType.DMA,
            pltpu.SemaphoreType.DMA,
            pltpu.SemaphoreType.DMA,
            pltpu.SemaphoreType.DMA,
            pltpu.SemaphoreType.DMA,
        ],
    )


def _sc_scatter_body(m7_hbm, s2_hbm, dst_hbm, z_hbm, agg_out,
                     s2b, db, rows, agg_sh,
                     isem0, isem1, isem2, isem3, gsem0, gsem1, ssem0, ssem1):
    c = lax.axis_index("c")
    s = lax.axis_index("s")
    wid = c * NS + s
    rows_per_tile = NP // NS
    isems = (isem0, isem1, isem2, isem3)
    gsems = (gsem0, gsem1)
    ssems = (ssem0, ssem1)

    def _idx_load(w, k):
        pltpu.async_copy(s2_hbm.at[wid, w], s2b.at[k], isems[k])
        pltpu.async_copy(dst_hbm.at[wid, w], db.at[k], isems[k])

    def _idx_wait(k):
        pltpu.make_async_copy(s2_hbm.at[wid, 0], s2b.at[k], isems[k]).wait()
        pltpu.make_async_copy(dst_hbm.at[wid, 0], db.at[k], isems[k]).wait()

    for k in range(3):
        _idx_load(k, k)
    pltpu.sync_copy(z_hbm, agg_sh.at[pl.ds(s * rows_per_tile, rows_per_tile)])
    _idx_wait(0)
    pltpu.async_copy(m7_hbm.at[s2b.at[0]], rows.at[0], gsem0)
    plsc.subcore_barrier()

    def _win(i, carry):
        for j in range(4):
            w = i * 4 + j
            b = j % 2
            pltpu.make_async_copy(
                m7_hbm.at[s2b.at[j]], rows.at[b], gsems[b]).wait()
            pltpu.async_copy(rows.at[b], agg_sh.at[db.at[j]],
                             ssems[b], add=True)

            @pl.when(w > 0)
            def _():
                pltpu.make_async_copy(
                    rows.at[1 - b], agg_sh.at[db.at[(j + 3) % 4]],
                    ssems[1 - b]).wait()

            @pl.when(w + 3 < NWIN)
            def _():
                _idx_load(w + 3, (j + 3) % 4)

            @pl.when(w + 1 < NWIN)
            def _():
                _idx_wait((j + 1) % 4)
                pltpu.async_copy(m7_hbm.at[s2b.at[(j + 1) % 4]],
                                 rows.at[1 - b], gsems[1 - b])
        return carry
    lax.fori_loop(0, NWIN // 4, _win, 0)
    pltpu.make_async_copy(rows.at[(NWIN - 1) % 2],
                          agg_sh.at[db.at[(NWIN - 1) % 4]],
                          ssems[(NWIN - 1) % 2]).wait()
    plsc.subcore_barrier()
    pltpu.sync_copy(agg_sh.at[pl.ds(s * rows_per_tile, rows_per_tile)],
                    agg_out.at[c, pl.ds(s * rows_per_tile, rows_per_tile)])


def _scaled_mm(ee, x, w):
    ew7 = jnp.mean(ee, axis=1)
    m = lax.dot_general(x, w, (((1,), (0,)), ((), ())),
                        preferred_element_type=jnp.float32, precision=_HI)
    return ew7[:, None, None] * m[None]


def _m7_body(ee_ref, w_ref, x_ref, out_ref):
    out_ref[...] = _scaled_mm(ee_ref[...], x_ref[...], w_ref[...])


def _gru(agg, x, wih, whh, bih, bhh):
    gi = lax.dot_general(agg, wih, (((1,), (1,)), ((), ())),
                         preferred_element_type=jnp.float32,
                         precision=_HI) + bih
    gh = lax.dot_general(x, whh, (((1,), (1,)), ((), ())),
                         preferred_element_type=jnp.float32,
                         precision=_HI) + bhh
    r = jax.nn.sigmoid(gi[:, :H] + gh[:, :H])
    z = jax.nn.sigmoid(gi[:, H:2 * H] + gh[:, H:2 * H])
    n = jnp.tanh(gi[:, 2 * H:] + r * gh[:, 2 * H:])
    return (1.0 - z) * n + z * x


def _ga_body(wih_ref, whh_ref, bih_ref, bhh_ref, ee_ref, wn_ref,
             agg_ref, x_ref, xo_ref, m7_ref):
    xn = _gru(agg_ref[0] + agg_ref[1], x_ref[...], wih_ref[...], whh_ref[...],
              bih_ref[...], bhh_ref[...])
    xo_ref[...] = xn
    m7_ref[...] = _scaled_mm(ee_ref[...], xn, wn_ref[...])


def _gru_body(wih_ref, whh_ref, bih_ref, bhh_ref, agg_ref, x_ref, xo_ref):
    xo_ref[...] = _gru(agg_ref[0] + agg_ref[1], x_ref[...], wih_ref[...],
                       whh_ref[...], bih_ref[...], bhh_ref[...])


def _pool_body(x_ref, gw_ref, gb_ref, o_ref):
    x = x_ref[...]
    raw = jnp.sum(x * gw_ref[...], axis=1, keepdims=True) + gb_ref[0, 0]
    g = jax.nn.sigmoid(raw)
    valid = lax.broadcasted_iota(jnp.int32, (NP, 1), 0) < N
    g = jnp.where(valid, g, 0.0)
    a = jnp.where(valid, jnp.exp(g - jnp.max(g)), 0.0)
    a = a / jnp.sum(a)
    o_ref[...] = lax.dot_general(a, x, (((0,), (0,)), ((), ())),
                                 preferred_element_type=jnp.float32,
                                 precision=_HI)


_full = pl.BlockSpec(index_map=lambda i: (0,) * 2)
_full3 = pl.BlockSpec(index_map=lambda i: (0,) * 3)


def _m7_call(ee, w, x):
    return pl.pallas_call(
        _m7_body,
        grid=(NP // BN,),
        in_specs=[pl.BlockSpec((7, H), lambda i: (0, 0)),
                  pl.BlockSpec((H, H), lambda i: (0, 0)),
                  pl.BlockSpec((BN, H), lambda i: (i, 0))],
        out_specs=pl.BlockSpec((7, BN, H), lambda i: (0, i, 0)),
        out_shape=jax.ShapeDtypeStruct((7, NP, H), jnp.float32),
    )(ee, w, x)


def _ga_call(wih, whh, bih, bhh, ee, wn, agg2, x):
    return pl.pallas_call(
        _ga_body,
        grid=(NP // BN,),
        in_specs=[pl.BlockSpec((3 * H, H), lambda i: (0, 0)),
                  pl.BlockSpec((3 * H, H), lambda i: (0, 0)),
                  pl.BlockSpec((1, 3 * H), lambda i: (0, 0)),
                  pl.BlockSpec((1, 3 * H), lambda i: (0, 0)),
                  pl.BlockSpec((7, H), lambda i: (0, 0)),
                  pl.BlockSpec((H, H), lambda i: (0, 0)),
                  pl.BlockSpec((NC, BN, H), lambda i: (0, i, 0)),
                  pl.BlockSpec((BN, H), lambda i: (i, 0))],
        out_specs=[pl.BlockSpec((BN, H), lambda i: (i, 0)),
                   pl.BlockSpec((7, BN, H), lambda i: (0, i, 0))],
        out_shape=[jax.ShapeDtypeStruct((NP, H), jnp.float32),
                   jax.ShapeDtypeStruct((7, NP, H), jnp.float32)],
    )(wih, whh, bih, bhh, ee, wn, agg2, x)


def _gru_call(wih, whh, bih, bhh, agg2, x):
    return pl.pallas_call(
        _gru_body,
        grid=(NP // BN,),
        in_specs=[pl.BlockSpec((3 * H, H), lambda i: (0, 0)),
                  pl.BlockSpec((3 * H, H), lambda i: (0, 0)),
                  pl.BlockSpec((1, 3 * H), lambda i: (0, 0)),
                  pl.BlockSpec((1, 3 * H), lambda i: (0, 0)),
                  pl.BlockSpec((NC, BN, H), lambda i: (0, i, 0)),
                  pl.BlockSpec((BN, H), lambda i: (i, 0))],
        out_specs=pl.BlockSpec((BN, H), lambda i: (i, 0)),
        out_shape=jax.ShapeDtypeStruct((NP, H), jnp.float32),
    )(wih, whh, bih, bhh, agg2, x)


def _pool_call(x, gw, gb):
    return pl.pallas_call(
        _pool_body,
        in_specs=[pl.BlockSpec((NP, H), lambda: (0, 0)),
                  pl.BlockSpec((1, H), lambda: (0, 0)),
                  pl.BlockSpec((1, 1), lambda: (0, 0))],
        out_specs=pl.BlockSpec((1, H), lambda: (0, 0)),
        out_shape=jax.ShapeDtypeStruct((1, H), jnp.float32),
    )(x, gw, gb)


def kernel(node_ids, edges, edge_types, embed_table, edge_embed_table,
           ggnn_weight, w_ih, w_hh, b_ih, b_hh, gate_w, gate_b):
    src = edges[0].astype(jnp.int32)
    dst = edges[1].astype(jnp.int32)
    et = edge_types.astype(jnp.int32)
    padi = jnp.arange(EP - E, dtype=jnp.int32) % 16
    src_p = jnp.concatenate([src, padi]).reshape(NW, EW)
    et_p = jnp.concatenate([et, jnp.ones((EP - E,), jnp.int32)]).reshape(NW, EW)
    dst_p = jnp.concatenate([dst, N + padi]).reshape(NW, NWIN, WSZ)
    ids_p = jnp.concatenate(
        [node_ids.astype(jnp.int32),
         jnp.arange(NP - N, dtype=jnp.int32) % 64]).reshape(NW, IDW)
    zrows = jnp.zeros((NP // NS, H), jnp.float32)
    bih = b_ih.reshape(1, 3 * H)
    bhh = b_hh.reshape(1, 3 * H)
    gb = gate_b.reshape(1, 1)

    x, s2 = _sc_embed_kernel()(embed_table, ids_p, src_p, et_p)
    s2 = s2.reshape(NW, NWIN, WSZ)
    m7 = _m7_call(edge_embed_table, ggnn_weight[0], x).reshape(7 * NP, H)
    for layer in range(L):
        agg2 = _sc_scatter_kernel()(m7, s2, dst_p, zrows)
        if layer + 1 < L:
            x, m7 = _ga_call(w_ih, w_hh, bih, bhh, edge_embed_table,
                             ggnn_weight[layer + 1], agg2, x)
            m7 = m7.reshape(7 * NP, H)
        else:
            x = _gru_call(w_ih, w_hh, bih, bhh, agg2, x)
    return _pool_call(x, gate_w, gb)


# speedup vs baseline: 13.2563x; 1.1981x over previous
import functools

import jax
import jax.numpy as jnp
from jax import lax
from jax.experimental import pallas as pl
from jax.experimental.pallas import tpu as pltpu
from jax.experimental.pallas import tpu_sc as plsc

N = 10000
NP = 10240
H = 128
E = 320000
EP = 327680
L = 4
NC = 2
NS = 16
NW = NC * NS
EW = EP // NW
NWIN = 160
WSZ = EW // NWIN
NRB = 4
NIB = 8
IDW = NP // NW
BN = 1024
_HI = lax.Precision.DEFAULT


@functools.cache
def _sc_embed_kernel():
    mesh = plsc.VectorSubcoreMesh(
        core_axis_name="c", subcore_axis_name="s",
        num_cores=NC, num_subcores=NS)
    return pl.kernel(
        _sc_embed_body,
        out_type=[jax.ShapeDtypeStruct((NP, H), jnp.float32),
                  jax.ShapeDtypeStruct((NW, EW), jnp.int32)],
        mesh=mesh,
        scratch_types=[
            pltpu.VMEM((IDW,), jnp.int32),
            pltpu.VMEM((IDW, H), jnp.float32),
            pltpu.VMEM((EW,), jnp.int32),
            pltpu.VMEM((EW,), jnp.int32),
            pltpu.SemaphoreType.DMA,
        ],
    )


def _sc_embed_body(emb_hbm, ids_hbm, src_hbm, et_hbm, x_out, s2_out,
                   idx_v, rows_v, sv, ev, sem):
    c = lax.axis_index("c")
    s = lax.axis_index("s")
    wid = c * NS + s
    pltpu.sync_copy(ids_hbm.at[wid], idx_v)

    def _inc(i, carry):
        sl = pl.ds(i * 16, 16)
        idx_v[sl] = idx_v[sl] + 1
        return carry
    lax.fori_loop(0, IDW // 16, _inc, 0)

    cps = []
    for j in range(IDW // 80):
        cps.append(pltpu.async_copy(
            emb_hbm.at[idx_v.at[pl.ds(j * 80, 80)]],
            rows_v.at[pl.ds(j * 80, 80)], sem))
    for cp in cps:
        cp.wait()
    pltpu.sync_copy(rows_v, x_out.at[pl.ds(wid * IDW, IDW)])
    pltpu.sync_copy(src_hbm.at[wid], sv)
    pltpu.sync_copy(et_hbm.at[wid], ev)

    def _s2(i, carry):
        sl = pl.ds(i * 16, 16)
        sv[sl] = (ev[sl] - 1) * NP + sv[sl]
        return carry
    lax.fori_loop(0, EW // 16, _s2, 0)
    pltpu.sync_copy(sv, s2_out.at[wid])


@functools.cache
def _sc_scatter_kernel():
    mesh = plsc.VectorSubcoreMesh(
        core_axis_name="c", subcore_axis_name="s",
        num_cores=NC, num_subcores=NS)
    return pl.kernel(
        _sc_scatter_body,
        out_type=jax.ShapeDtypeStruct((NC, NP, H), jnp.float32),
        mesh=mesh,
        scratch_types=[
            pltpu.VMEM((NIB, WSZ), jnp.int32),
            pltpu.VMEM((NIB, WSZ), jnp.int32),
            pltpu.VMEM((NRB, WSZ, H), jnp.float32),
            pltpu.VMEM_SHARED((NP, H), jnp.float32),
        ] + [pltpu.SemaphoreType.DMA] * (NIB + 2 * NRB),
    )


def _sc_scatter_body(m7_hbm, s2_hbm, dst_hbm, z_hbm, agg_out,
                     s2b, db, rows, agg_sh, *sems):
    c = lax.axis_index("c")
    s = lax.axis_index("s")
    wid = c * NS + s
    rows_per_tile = NP // NS
    isems = sems[:NIB]
    gsems = sems[NIB:NIB + NRB]
    ssems = sems[NIB + NRB:]

    def _idx_load(w, k):
        pltpu.async_copy(s2_hbm.at[wid, w], s2b.at[k], isems[k])
        pltpu.async_copy(dst_hbm.at[wid, w], db.at[k], isems[k])

    def _idx_wait(k):
        pltpu.make_async_copy(s2_hbm.at[wid, 0], s2b.at[k], isems[k]).wait()
        pltpu.make_async_copy(dst_hbm.at[wid, 0], db.at[k], isems[k]).wait()

    def _gather(k, b):
        pltpu.async_copy(m7_hbm.at[s2b.at[k]], rows.at[b], gsems[b])

    for k in range(5):
        _idx_load(k, k)
    pltpu.sync_copy(z_hbm, agg_sh.at[pl.ds(s * rows_per_tile, rows_per_tile)])
    for k in range(NRB - 1):
        _idx_wait(k)
        _gather(k, k)
    plsc.subcore_barrier()

    def _win(i, carry):
        for j in range(NIB):
            w = i * NIB + j
            b = j % NRB
            pltpu.make_async_copy(
                m7_hbm.at[s2b.at[j]], rows.at[b], gsems[b]).wait()
            pltpu.async_copy(rows.at[b], agg_sh.at[db.at[j]],
                             ssems[b], add=True)

            @pl.when(w > 0)
            def _():
                pltpu.make_async_copy(
                    rows.at[(b + 3) % NRB], agg_sh.at[db.at[(j + 7) % NIB]],
                    ssems[(b + 3) % NRB]).wait()

            @pl.when(w + 3 < NWIN)
            def _():
                _idx_wait((j + 3) % NIB)
                _gather((j + 3) % NIB, (b + 3) % NRB)

            @pl.when(w + 5 < NWIN)
            def _():
                _idx_load(w + 5, (j + 5) % NIB)
        return carry
    lax.fori_loop(0, NWIN // NIB, _win, 0)
    pltpu.make_async_copy(rows.at[(NWIN - 1) % NRB],
                          agg_sh.at[db.at[(NWIN - 1) % NIB]],
                          ssems[(NWIN - 1) % NRB]).wait()
    plsc.subcore_barrier()
    pltpu.sync_copy(agg_sh.at[pl.ds(s * rows_per_tile, rows_per_tile)],
                    agg_out.at[c, pl.ds(s * rows_per_tile, rows_per_tile)])


def _scaled_mm(ee, x, w):
    ew7 = jnp.mean(ee, axis=1)
    m = lax.dot_general(x, w, (((1,), (0,)), ((), ())),
                        preferred_element_type=jnp.float32, precision=_HI)
    return ew7[:, None, None] * m[None]


def _m7_body(ee_ref, w_ref, x_ref, out_ref):
    out_ref[...] = _scaled_mm(ee_ref[...], x_ref[...], w_ref[...])


def _gru(agg, x, wih, whh, bih, bhh):
    gi = lax.dot_general(agg, wih, (((1,), (1,)), ((), ())),
                         preferred_element_type=jnp.float32,
                         precision=_HI) + bih
    gh = lax.dot_general(x, whh, (((1,), (1,)), ((), ())),
                         preferred_element_type=jnp.float32,
                         precision=_HI) + bhh
    r = jax.nn.sigmoid(gi[:, :H] + gh[:, :H])
    z = jax.nn.sigmoid(gi[:, H:2 * H] + gh[:, H:2 * H])
    n = jnp.tanh(gi[:, 2 * H:] + r * gh[:, 2 * H:])
    return (1.0 - z) * n + z * x


def _ga_body(wih_ref, whh_ref, bih_ref, bhh_ref, ee_ref, wn_ref,
             agg_ref, x_ref, xo_ref, m7_ref):
    xn = _gru(agg_ref[0] + agg_ref[1], x_ref[...], wih_ref[...], whh_ref[...],
              bih_ref[...], bhh_ref[...])
    xo_ref[...] = xn
    m7_ref[...] = _scaled_mm(ee_ref[...], xn, wn_ref[...])


def _gru_body(wih_ref, whh_ref, bih_ref, bhh_ref, agg_ref, x_ref, xo_ref):
    xo_ref[...] = _gru(agg_ref[0] + agg_ref[1], x_ref[...], wih_ref[...],
                       whh_ref[...], bih_ref[...], bhh_ref[...])


def _pool_body(x_ref, gw_ref, gb_ref, o_ref):
    x = x_ref[...]
    raw = jnp.sum(x * gw_ref[...], axis=1, keepdims=True) + gb_ref[0, 0]
    g = jax.nn.sigmoid(raw)
    valid = lax.broadcasted_iota(jnp.int32, (NP, 1), 0) < N
    g = jnp.where(valid, g, 0.0)
    a = jnp.where(valid, jnp.exp(g - jnp.max(g)), 0.0)
    a = a / jnp.sum(a)
    o_ref[...] = lax.dot_general(a, x, (((0,), (0,)), ((), ())),
                                 preferred_element_type=jnp.float32,
                                 precision=_HI)


_full = pl.BlockSpec(index_map=lambda i: (0,) * 2)
_full3 = pl.BlockSpec(index_map=lambda i: (0,) * 3)


def _m7_call(ee, w, x):
    return pl.pallas_call(
        _m7_body,
        grid=(NP // BN,),
        in_specs=[pl.BlockSpec((7, H), lambda i: (0, 0)),
                  pl.BlockSpec((H, H), lambda i: (0, 0)),
                  pl.BlockSpec((BN, H), lambda i: (i, 0))],
        out_specs=pl.BlockSpec((7, BN, H), lambda i: (0, i, 0)),
        out_shape=jax.ShapeDtypeStruct((7, NP, H), jnp.float32),
    )(ee, w, x)


def _ga_call(wih, whh, bih, bhh, ee, wn, agg2, x):
    return pl.pallas_call(
        _ga_body,
        grid=(NP // BN,),
        in_specs=[pl.BlockSpec((3 * H, H), lambda i: (0, 0)),
                  pl.BlockSpec((3 * H, H), lambda i: (0, 0)),
                  pl.BlockSpec((1, 3 * H), lambda i: (0, 0)),
                  pl.BlockSpec((1, 3 * H), lambda i: (0, 0)),
                  pl.BlockSpec((7, H), lambda i: (0, 0)),
                  pl.BlockSpec((H, H), lambda i: (0, 0)),
                  pl.BlockSpec((NC, BN, H), lambda i: (0, i, 0)),
                  pl.BlockSpec((BN, H), lambda i: (i, 0))],
        out_specs=[pl.BlockSpec((BN, H), lambda i: (i, 0)),
                   pl.BlockSpec((7, BN, H), lambda i: (0, i, 0))],
        out_shape=[jax.ShapeDtypeStruct((NP, H), jnp.float32),
                   jax.ShapeDtypeStruct((7, NP, H), jnp.float32)],
    )(wih, whh, bih, bhh, ee, wn, agg2, x)


def _gru_call(wih, whh, bih, bhh, agg2, x):
    return pl.pallas_call(
        _gru_body,
        grid=(NP // BN,),
        in_specs=[pl.BlockSpec((3 * H, H), lambda i: (0, 0)),
                  pl.BlockSpec((3 * H, H), lambda i: (0, 0)),
                  pl.BlockSpec((1, 3 * H), lambda i: (0, 0)),
                  pl.BlockSpec((1, 3 * H), lambda i: (0, 0)),
                  pl.BlockSpec((NC, BN, H), lambda i: (0, i, 0)),
                  pl.BlockSpec((BN, H), lambda i: (i, 0))],
        out_specs=pl.BlockSpec((BN, H), lambda i: (i, 0)),
        out_shape=jax.ShapeDtypeStruct((NP, H), jnp.float32),
    )(wih, whh, bih, bhh, agg2, x)


def _pool_call(x, gw, gb):
    return pl.pallas_call(
        _pool_body,
        in_specs=[pl.BlockSpec((NP, H), lambda: (0, 0)),
                  pl.BlockSpec((1, H), lambda: (0, 0)),
                  pl.BlockSpec((1, 1), lambda: (0, 0))],
        out_specs=pl.BlockSpec((1, H), lambda: (0, 0)),
        out_shape=jax.ShapeDtypeStruct((1, H), jnp.float32),
    )(x, gw, gb)


def kernel(node_ids, edges, edge_types, embed_table, edge_embed_table,
           ggnn_weight, w_ih, w_hh, b_ih, b_hh, gate_w, gate_b):
    src = edges[0].astype(jnp.int32)
    dst = edges[1].astype(jnp.int32)
    et = edge_types.astype(jnp.int32)
    padi = jnp.arange(EP - E, dtype=jnp.int32) % 16
    src_p = jnp.concatenate([src, padi]).reshape(NW, EW)
    et_p = jnp.concatenate([et, jnp.ones((EP - E,), jnp.int32)]).reshape(NW, EW)
    dst_p = jnp.concatenate([dst, N + padi]).reshape(NW, NWIN, WSZ)
    ids_p = jnp.concatenate(
        [node_ids.astype(jnp.int32),
         jnp.arange(NP - N, dtype=jnp.int32) % 64]).reshape(NW, IDW)
    zrows = jnp.zeros((NP // NS, H), jnp.float32)
    bih = b_ih.reshape(1, 3 * H)
    bhh = b_hh.reshape(1, 3 * H)
    gb = gate_b.reshape(1, 1)

    x, s2 = _sc_embed_kernel()(embed_table, ids_p, src_p, et_p)
    s2 = s2.reshape(NW, NWIN, WSZ)
    m7 = _m7_call(edge_embed_table, ggnn_weight[0], x).reshape(7 * NP, H)
    for layer in range(L):
        agg2 = _sc_scatter_kernel()(m7, s2, dst_p, zrows)
        if layer + 1 < L:
            x, m7 = _ga_call(w_ih, w_hh, bih, bhh, edge_embed_table,
                             ggnn_weight[layer + 1], agg2, x)
            m7 = m7.reshape(7 * NP, H)
        else:
            x = _gru_call(w_ih, w_hh, bih, bhh, agg2, x)
    return _pool_call(x, gate_w, gb)


# speedup vs baseline: 13.5528x; 1.0224x over previous
import functools

import jax
import jax.numpy as jnp
from jax import lax
from jax.experimental import pallas as pl
from jax.experimental.pallas import tpu as pltpu
from jax.experimental.pallas import tpu_sc as plsc

N = 10000
NP = 10240
H = 128
E = 320000
EP = 327680
L = 4
NC = 2
NS = 16
NW = NC * NS
EW = EP // NW
NWIN = 160
WSZ = EW // NWIN
NRB = 5
NIB = 10
GLA = NRB - 1
ILA = GLA + 2
IDW = NP // NW
BN = 1024
_HI = lax.Precision.DEFAULT


@functools.cache
def _sc_embed_kernel():
    mesh = plsc.VectorSubcoreMesh(
        core_axis_name="c", subcore_axis_name="s",
        num_cores=NC, num_subcores=NS)
    return pl.kernel(
        _sc_embed_body,
        out_type=[jax.ShapeDtypeStruct((NP, H), jnp.float32),
                  jax.ShapeDtypeStruct((NW, EW), jnp.int32)],
        mesh=mesh,
        scratch_types=[
            pltpu.VMEM((IDW,), jnp.int32),
            pltpu.VMEM((IDW, H), jnp.float32),
            pltpu.VMEM((EW,), jnp.int32),
            pltpu.VMEM((EW,), jnp.int32),
            pltpu.SemaphoreType.DMA,
        ],
    )


def _sc_embed_body(emb_hbm, ids_hbm, src_hbm, et_hbm, x_out, s2_out,
                   idx_v, rows_v, sv, ev, sem):
    c = lax.axis_index("c")
    s = lax.axis_index("s")
    wid = c * NS + s
    pltpu.sync_copy(ids_hbm.at[wid], idx_v)

    def _inc(i, carry):
        sl = pl.ds(i * 16, 16)
        idx_v[sl] = idx_v[sl] + 1
        return carry
    lax.fori_loop(0, IDW // 16, _inc, 0)

    cps = []
    for j in range(IDW // 80):
        cps.append(pltpu.async_copy(
            emb_hbm.at[idx_v.at[pl.ds(j * 80, 80)]],
            rows_v.at[pl.ds(j * 80, 80)], sem))
    for cp in cps:
        cp.wait()
    pltpu.sync_copy(rows_v, x_out.at[pl.ds(wid * IDW, IDW)])
    pltpu.sync_copy(src_hbm.at[wid], sv)
    pltpu.sync_copy(et_hbm.at[wid], ev)

    def _s2(i, carry):
        sl = pl.ds(i * 16, 16)
        sv[sl] = (ev[sl] - 1) * NP + sv[sl]
        return carry
    lax.fori_loop(0, EW // 16, _s2, 0)
    pltpu.sync_copy(sv, s2_out.at[wid])


@functools.cache
def _sc_scatter_kernel():
    mesh = plsc.VectorSubcoreMesh(
        core_axis_name="c", subcore_axis_name="s",
        num_cores=NC, num_subcores=NS)
    return pl.kernel(
        _sc_scatter_body,
        out_type=jax.ShapeDtypeStruct((NC, NP, H), jnp.float32),
        mesh=mesh,
        scratch_types=[
            pltpu.VMEM((NIB, WSZ), jnp.int32),
            pltpu.VMEM((NIB, WSZ), jnp.int32),
            pltpu.VMEM((NRB, WSZ, H), jnp.float32),
            pltpu.VMEM_SHARED((NP, H), jnp.float32),
        ] + [pltpu.SemaphoreType.DMA] * (NIB + 2 * NRB),
    )


def _sc_scatter_body(m7_hbm, s2_hbm, dst_hbm, z_hbm, agg_out,
                     s2b, db, rows, agg_sh, *sems):
    c = lax.axis_index("c")
    s = lax.axis_index("s")
    wid = c * NS + s
    rows_per_tile = NP // NS
    isems = sems[:NIB]
    gsems = sems[NIB:NIB + NRB]
    ssems = sems[NIB + NRB:]

    def _idx_load(w, k):
        pltpu.async_copy(s2_hbm.at[wid, w], s2b.at[k], isems[k])
        pltpu.async_copy(dst_hbm.at[wid, w], db.at[k], isems[k])

    def _idx_wait(k):
        pltpu.make_async_copy(s2_hbm.at[wid, 0], s2b.at[k], isems[k]).wait()
        pltpu.make_async_copy(dst_hbm.at[wid, 0], db.at[k], isems[k]).wait()

    def _gather(k, b):
        pltpu.async_copy(m7_hbm.at[s2b.at[k]], rows.at[b], gsems[b])

    for k in range(ILA):
        _idx_load(k, k)
    pltpu.sync_copy(z_hbm, agg_sh.at[pl.ds(s * rows_per_tile, rows_per_tile)])
    for k in range(GLA):
        _idx_wait(k)
        _gather(k, k)
    plsc.subcore_barrier()

    def _win(i, carry):
        for j in range(NIB):
            w = i * NIB + j
            b = j % NRB
            pltpu.make_async_copy(
                m7_hbm.at[s2b.at[j]], rows.at[b], gsems[b]).wait()
            pltpu.async_copy(rows.at[b], agg_sh.at[db.at[j]],
                             ssems[b], add=True)

            @pl.when(w > 0)
            def _():
                pltpu.make_async_copy(
                    rows.at[(b + NRB - 1) % NRB],
                    agg_sh.at[db.at[(j + NIB - 1) % NIB]],
                    ssems[(b + NRB - 1) % NRB]).wait()

            @pl.when(w + GLA < NWIN)
            def _():
                _idx_wait((j + GLA) % NIB)
                _gather((j + GLA) % NIB, (b + GLA) % NRB)

            @pl.when(w + ILA < NWIN)
            def _():
                _idx_load(w + ILA, (j + ILA) % NIB)
        return carry
    lax.fori_loop(0, NWIN // NIB, _win, 0)
    pltpu.make_async_copy(rows.at[(NWIN - 1) % NRB],
                          agg_sh.at[db.at[(NWIN - 1) % NIB]],
                          ssems[(NWIN - 1) % NRB]).wait()
    plsc.subcore_barrier()
    pltpu.sync_copy(agg_sh.at[pl.ds(s * rows_per_tile, rows_per_tile)],
                    agg_out.at[c, pl.ds(s * rows_per_tile, rows_per_tile)])


def _scaled_mm(ee, x, w):
    ew7 = jnp.mean(ee, axis=1)
    m = lax.dot_general(x, w, (((1,), (0,)), ((), ())),
                        preferred_element_type=jnp.float32, precision=_HI)
    return ew7[:, None, None] * m[None]


def _m7_body(ee_ref, w_ref, x_ref, out_ref):
    out_ref[...] = _scaled_mm(ee_ref[...], x_ref[...], w_ref[...])


def _gru(agg, x, wih, whh, bih, bhh):
    gi = lax.dot_general(agg, wih, (((1,), (1,)), ((), ())),
                         preferred_element_type=jnp.float32,
                         precision=_HI) + bih
    gh = lax.dot_general(x, whh, (((1,), (1,)), ((), ())),
                         preferred_element_type=jnp.float32,
                         precision=_HI) + bhh
    r = jax.nn.sigmoid(gi[:, :H] + gh[:, :H])
    z = jax.nn.sigmoid(gi[:, H:2 * H] + gh[:, H:2 * H])
    n = jnp.tanh(gi[:, 2 * H:] + r * gh[:, 2 * H:])
    return (1.0 - z) * n + z * x


def _ga_body(wih_ref, whh_ref, bih_ref, bhh_ref, ee_ref, wn_ref,
             agg_ref, x_ref, xo_ref, m7_ref):
    xn = _gru(agg_ref[0] + agg_ref[1], x_ref[...], wih_ref[...], whh_ref[...],
              bih_ref[...], bhh_ref[...])
    xo_ref[...] = xn
    m7_ref[...] = _scaled_mm(ee_ref[...], xn, wn_ref[...])


def _gru_body(wih_ref, whh_ref, bih_ref, bhh_ref, agg_ref, x_ref, xo_ref):
    xo_ref[...] = _gru(agg_ref[0] + agg_ref[1], x_ref[...], wih_ref[...],
                       whh_ref[...], bih_ref[...], bhh_ref[...])


def _pool_body(x_ref, gw_ref, gb_ref, o_ref):
    x = x_ref[...]
    raw = jnp.sum(x * gw_ref[...], axis=1, keepdims=True) + gb_ref[0, 0]
    g = jax.nn.sigmoid(raw)
    valid = lax.broadcasted_iota(jnp.int32, (NP, 1), 0) < N
    g = jnp.where(valid, g, 0.0)
    a = jnp.where(valid, jnp.exp(g - jnp.max(g)), 0.0)
    a = a / jnp.sum(a)
    o_ref[...] = lax.dot_general(a, x, (((0,), (0,)), ((), ())),
                                 preferred_element_type=jnp.float32,
                                 precision=_HI)


_full = pl.BlockSpec(index_map=lambda i: (0,) * 2)
_full3 = pl.BlockSpec(index_map=lambda i: (0,) * 3)


def _m7_call(ee, w, x):
    return pl.pallas_call(
        _m7_body,
        grid=(NP // BN,),
        in_specs=[pl.BlockSpec((7, H), lambda i: (0, 0)),
                  pl.BlockSpec((H, H), lambda i: (0, 0)),
                  pl.BlockSpec((BN, H), lambda i: (i, 0))],
        out_specs=pl.BlockSpec((7, BN, H), lambda i: (0, i, 0)),
        out_shape=jax.ShapeDtypeStruct((7, NP, H), jnp.float32),
    )(ee, w, x)


def _ga_call(wih, whh, bih, bhh, ee, wn, agg2, x):
    return pl.pallas_call(
        _ga_body,
        grid=(NP // BN,),
        in_specs=[pl.BlockSpec((3 * H, H), lambda i: (0, 0)),
                  pl.BlockSpec((3 * H, H), lambda i: (0, 0)),
                  pl.BlockSpec((1, 3 * H), lambda i: (0, 0)),
                  pl.BlockSpec((1, 3 * H), lambda i: (0, 0)),
                  pl.BlockSpec((7, H), lambda i: (0, 0)),
                  pl.BlockSpec((H, H), lambda i: (0, 0)),
                  pl.BlockSpec((NC, BN, H), lambda i: (0, i, 0)),
                  pl.BlockSpec((BN, H), lambda i: (i, 0))],
        out_specs=[pl.BlockSpec((BN, H), lambda i: (i, 0)),
                   pl.BlockSpec((7, BN, H), lambda i: (0, i, 0))],
        out_shape=[jax.ShapeDtypeStruct((NP, H), jnp.float32),
                   jax.ShapeDtypeStruct((7, NP, H), jnp.float32)],
    )(wih, whh, bih, bhh, ee, wn, agg2, x)


def _gru_call(wih, whh, bih, bhh, agg2, x):
    return pl.pallas_call(
        _gru_body,
        grid=(NP // BN,),
        in_specs=[pl.BlockSpec((3 * H, H), lambda i: (0, 0)),
                  pl.BlockSpec((3 * H, H), lambda i: (0, 0)),
                  pl.BlockSpec((1, 3 * H), lambda i: (0, 0)),
                  pl.BlockSpec((1, 3 * H), lambda i: (0, 0)),
                  pl.BlockSpec((NC, BN, H), lambda i: (0, i, 0)),
                  pl.BlockSpec((BN, H), lambda i: (i, 0))],
        out_specs=pl.BlockSpec((BN, H), lambda i: (i, 0)),
        out_shape=jax.ShapeDtypeStruct((NP, H), jnp.float32),
    )(wih, whh, bih, bhh, agg2, x)


def _pool_call(x, gw, gb):
    return pl.pallas_call(
        _pool_body,
        in_specs=[pl.BlockSpec((NP, H), lambda: (0, 0)),
                  pl.BlockSpec((1, H), lambda: (0, 0)),
                  pl.BlockSpec((1, 1), lambda: (0, 0))],
        out_specs=pl.BlockSpec((1, H), lambda: (0, 0)),
        out_shape=jax.ShapeDtypeStruct((1, H), jnp.float32),
    )(x, gw, gb)


def kernel(node_ids, edges, edge_types, embed_table, edge_embed_table,
           ggnn_weight, w_ih, w_hh, b_ih, b_hh, gate_w, gate_b):
    src = edges[0].astype(jnp.int32)
    dst = edges[1].astype(jnp.int32)
    et = edge_types.astype(jnp.int32)
    padi = jnp.arange(EP - E, dtype=jnp.int32) % 16
    src_p = jnp.concatenate([src, padi]).reshape(NW, EW)
    et_p = jnp.concatenate([et, jnp.ones((EP - E,), jnp.int32)]).reshape(NW, EW)
    dst_p = jnp.concatenate([dst, N + padi]).reshape(NW, NWIN, WSZ)
    ids_p = jnp.concatenate(
        [node_ids.astype(jnp.int32),
         jnp.arange(NP - N, dtype=jnp.int32) % 64]).reshape(NW, IDW)
    zrows = jnp.zeros((NP // NS, H), jnp.float32)
    bih = b_ih.reshape(1, 3 * H)
    bhh = b_hh.reshape(1, 3 * H)
    gb = gate_b.reshape(1, 1)

    x, s2 = _sc_embed_kernel()(embed_table, ids_p, src_p, et_p)
    s2 = s2.reshape(NW, NWIN, WSZ)
    m7 = _m7_call(edge_embed_table, ggnn_weight[0], x).reshape(7 * NP, H)
    for layer in range(L):
        agg2 = _sc_scatter_kernel()(m7, s2, dst_p, zrows)
        if layer + 1 < L:
            x, m7 = _ga_call(w_ih, w_hh, bih, bhh, edge_embed_table,
                             ggnn_weight[layer + 1], agg2, x)
            m7 = m7.reshape(7 * NP, H)
        else:
            x = _gru_call(w_ih, w_hh, bih, bhh, agg2, x)
    return _pool_call(x, gate_w, gb)


# speedup vs baseline: 13.6883x; 1.0100x over previous
import functools

import jax
import jax.numpy as jnp
from jax import lax
from jax.experimental import pallas as pl
from jax.experimental.pallas import tpu as pltpu
from jax.experimental.pallas import tpu_sc as plsc

N = 10000
NP = 10240
H = 128
E = 320000
EP = 327680
L = 4
NC = 2
NS = 16
NW = NC * NS
EW = EP // NW
NWIN = 160
WSZ = EW // NWIN
NRB = 5
NIB = 10
GLA = NRB - 1
ILA = GLA + 2
IDW = NP // NW
BN = 1024
_HI = lax.Precision.DEFAULT


@functools.cache
def _sc_embed_kernel():
    mesh = plsc.VectorSubcoreMesh(
        core_axis_name="c", subcore_axis_name="s",
        num_cores=NC, num_subcores=NS)
    return pl.kernel(
        _sc_embed_body,
        out_type=[jax.ShapeDtypeStruct((NP, H), jnp.float32),
                  jax.ShapeDtypeStruct((NW, EW), jnp.int32)],
        mesh=mesh,
        scratch_types=[
            pltpu.VMEM((IDW,), jnp.int32),
            pltpu.VMEM((IDW, H), jnp.float32),
            pltpu.VMEM((EW,), jnp.int32),
            pltpu.VMEM((EW,), jnp.int32),
            pltpu.SemaphoreType.DMA,
        ],
    )


def _sc_embed_body(emb_hbm, ids_hbm, src_hbm, et_hbm, x_out, s2_out,
                   idx_v, rows_v, sv, ev, sem):
    c = lax.axis_index("c")
    s = lax.axis_index("s")
    wid = c * NS + s
    pltpu.sync_copy(ids_hbm.at[wid], idx_v)

    def _inc(i, carry):
        sl = pl.ds(i * 16, 16)
        idx_v[sl] = idx_v[sl] + 1
        return carry
    lax.fori_loop(0, IDW // 16, _inc, 0)

    cps = []
    for j in range(IDW // 80):
        cps.append(pltpu.async_copy(
            emb_hbm.at[idx_v.at[pl.ds(j * 80, 80)]],
            rows_v.at[pl.ds(j * 80, 80)], sem))
    for cp in cps:
        cp.wait()
    pltpu.sync_copy(rows_v, x_out.at[pl.ds(wid * IDW, IDW)])
    pltpu.sync_copy(src_hbm.at[wid], sv)
    pltpu.sync_copy(et_hbm.at[wid], ev)

    def _s2(i, carry):
        sl = pl.ds(i * 16, 16)
        sv[sl] = (ev[sl] - 1) * NP + sv[sl]
        return carry
    lax.fori_loop(0, EW // 16, _s2, 0)
    pltpu.sync_copy(sv, s2_out.at[wid])


@functools.cache
def _sc_scatter_kernel():
    mesh = plsc.VectorSubcoreMesh(
        core_axis_name="c", subcore_axis_name="s",
        num_cores=NC, num_subcores=NS)
    return pl.kernel(
        _sc_scatter_body,
        out_type=jax.ShapeDtypeStruct((NC, NP, H), jnp.float32),
        mesh=mesh,
        scratch_types=[
            pltpu.VMEM((NIB, WSZ), jnp.int32),
            pltpu.VMEM((NIB, WSZ), jnp.int32),
            pltpu.VMEM((NRB, WSZ, H), jnp.float32),
            pltpu.VMEM_SHARED((NP, H), jnp.float32),
        ] + [pltpu.SemaphoreType.DMA] * (NIB + 2 * NRB),
    )


def _sc_scatter_body(m7_hbm, s2_hbm, dst_hbm, z_hbm, agg_out,
                     s2b, db, rows, agg_sh, *sems):
    c = lax.axis_index("c")
    s = lax.axis_index("s")
    wid = c * NS + s
    rows_per_tile = NP // NS
    isems = sems[:NIB]
    gsems = sems[NIB:NIB + NRB]
    ssems = sems[NIB + NRB:]

    def _idx_load(w, k):
        pltpu.async_copy(s2_hbm.at[wid, w], s2b.at[k], isems[k])
        pltpu.async_copy(dst_hbm.at[wid, w], db.at[k], isems[k])

    def _idx_wait(k):
        pltpu.make_async_copy(s2_hbm.at[wid, 0], s2b.at[k], isems[k]).wait()
        pltpu.make_async_copy(dst_hbm.at[wid, 0], db.at[k], isems[k]).wait()

    def _gather(k, b):
        pltpu.async_copy(m7_hbm.at[s2b.at[k]], rows.at[b], gsems[b])

    for k in range(ILA):
        _idx_load(k, k)
    pltpu.sync_copy(z_hbm, agg_sh.at[pl.ds(s * rows_per_tile, rows_per_tile)])
    for k in range(GLA):
        _idx_wait(k)
        _gather(k, k)
    plsc.subcore_barrier()

    def _win(i, carry):
        for j in range(NIB):
            w = i * NIB + j
            b = j % NRB
            pltpu.make_async_copy(
                m7_hbm.at[s2b.at[j]], rows.at[b], gsems[b]).wait()
            pltpu.async_copy(rows.at[b], agg_sh.at[db.at[j]],
                             ssems[b], add=True)

            @pl.when(w > 0)
            def _():
                pltpu.make_async_copy(
                    rows.at[(b + NRB - 1) % NRB],
                    agg_sh.at[db.at[(j + NIB - 1) % NIB]],
                    ssems[(b + NRB - 1) % NRB]).wait()

            @pl.when(w + GLA < NWIN)
            def _():
                _idx_wait((j + GLA) % NIB)
                _gather((j + GLA) % NIB, (b + GLA) % NRB)

            @pl.when(w + ILA < NWIN)
            def _():
                _idx_load(w + ILA, (j + ILA) % NIB)
        return carry
    lax.fori_loop(0, NWIN // NIB, _win, 0)
    pltpu.make_async_copy(rows.at[(NWIN - 1) % NRB],
                          agg_sh.at[db.at[(NWIN - 1) % NIB]],
                          ssems[(NWIN - 1) % NRB]).wait()
    plsc.subcore_barrier()
    pltpu.sync_copy(agg_sh.at[pl.ds(s * rows_per_tile, rows_per_tile)],
                    agg_out.at[c, pl.ds(s * rows_per_tile, rows_per_tile)])


def _scaled_mm(ee, x, w):
    ew7 = jnp.mean(ee, axis=1)
    m = lax.dot_general(x, w, (((1,), (0,)), ((), ())),
                        preferred_element_type=jnp.float32, precision=_HI)
    return ew7[:, None, None] * m[None]


def _m7_body(ee_ref, w_ref, x_ref, out_ref):
    out_ref[...] = _scaled_mm(ee_ref[...], x_ref[...], w_ref[...])


def _gru(agg, x, wih, whh, bih, bhh):
    gi = lax.dot_general(agg, wih, (((1,), (1,)), ((), ())),
                         preferred_element_type=jnp.float32,
                         precision=_HI) + bih
    gh = lax.dot_general(x, whh, (((1,), (1,)), ((), ())),
                         preferred_element_type=jnp.float32,
                         precision=_HI) + bhh
    r = jax.nn.sigmoid(gi[:, :H] + gh[:, :H])
    z = jax.nn.sigmoid(gi[:, H:2 * H] + gh[:, H:2 * H])
    n = jnp.tanh(gi[:, 2 * H:] + r * gh[:, 2 * H:])
    return (1.0 - z) * n + z * x


def _ga_body(wih_ref, whh_ref, bih_ref, bhh_ref, ee_ref, wn_ref,
             agg_ref, x_ref, xo_ref, m7_ref):
    xn = _gru(agg_ref[0] + agg_ref[1], x_ref[...], wih_ref[...], whh_ref[...],
              bih_ref[...], bhh_ref[...])
    xo_ref[...] = xn
    m7_ref[...] = _scaled_mm(ee_ref[...], xn, wn_ref[...])


def _gru_pool_body(wih_ref, whh_ref, bih_ref, bhh_ref, gw_ref, gb_ref,
                   agg_ref, x_ref, o_ref):
    x = _gru(agg_ref[0] + agg_ref[1], x_ref[...], wih_ref[...],
             whh_ref[...], bih_ref[...], bhh_ref[...])
    raw = jnp.sum(x * gw_ref[...], axis=1, keepdims=True) + gb_ref[0, 0]
    g = jax.nn.sigmoid(raw)
    valid = lax.broadcasted_iota(jnp.int32, (NP, 1), 0) < N
    g = jnp.where(valid, g, 0.0)
    a = jnp.where(valid, jnp.exp(g - jnp.max(g)), 0.0)
    a = a / jnp.sum(a)
    o_ref[...] = lax.dot_general(a, x, (((0,), (0,)), ((), ())),
                                 preferred_element_type=jnp.float32,
                                 precision=_HI)


_full = pl.BlockSpec(index_map=lambda i: (0,) * 2)
_full3 = pl.BlockSpec(index_map=lambda i: (0,) * 3)


def _m7_call(ee, w, x):
    return pl.pallas_call(
        _m7_body,
        grid=(NP // BN,),
        in_specs=[pl.BlockSpec((7, H), lambda i: (0, 0)),
                  pl.BlockSpec((H, H), lambda i: (0, 0)),
                  pl.BlockSpec((BN, H), lambda i: (i, 0))],
        out_specs=pl.BlockSpec((7, BN, H), lambda i: (0, i, 0)),
        out_shape=jax.ShapeDtypeStruct((7, NP, H), jnp.float32),
    )(ee, w, x)


def _ga_call(wih, whh, bih, bhh, ee, wn, agg2, x):
    return pl.pallas_call(
        _ga_body,
        grid=(NP // BN,),
        in_specs=[pl.BlockSpec((3 * H, H), lambda i: (0, 0)),
                  pl.BlockSpec((3 * H, H), lambda i: (0, 0)),
                  pl.BlockSpec((1, 3 * H), lambda i: (0, 0)),
                  pl.BlockSpec((1, 3 * H), lambda i: (0, 0)),
                  pl.BlockSpec((7, H), lambda i: (0, 0)),
                  pl.BlockSpec((H, H), lambda i: (0, 0)),
                  pl.BlockSpec((NC, BN, H), lambda i: (0, i, 0)),
                  pl.BlockSpec((BN, H), lambda i: (i, 0))],
        out_specs=[pl.BlockSpec((BN, H), lambda i: (i, 0)),
                   pl.BlockSpec((7, BN, H), lambda i: (0, i, 0))],
        out_shape=[jax.ShapeDtypeStruct((NP, H), jnp.float32),
                   jax.ShapeDtypeStruct((7, NP, H), jnp.float32)],
    )(wih, whh, bih, bhh, ee, wn, agg2, x)


def _gru_pool_call(wih, whh, bih, bhh, gw, gb, agg2, x):
    return pl.pallas_call(
        _gru_pool_body,
        in_specs=[pl.BlockSpec((3 * H, H), lambda: (0, 0)),
                  pl.BlockSpec((3 * H, H), lambda: (0, 0)),
                  pl.BlockSpec((1, 3 * H), lambda: (0, 0)),
                  pl.BlockSpec((1, 3 * H), lambda: (0, 0)),
                  pl.BlockSpec((1, H), lambda: (0, 0)),
                  pl.BlockSpec((1, 1), lambda: (0, 0)),
                  pl.BlockSpec((NC, NP, H), lambda: (0, 0, 0)),
                  pl.BlockSpec((NP, H), lambda: (0, 0))],
        out_specs=pl.BlockSpec((1, H), lambda: (0, 0)),
        out_shape=jax.ShapeDtypeStruct((1, H), jnp.float32),
    )(wih, whh, bih, bhh, gw, gb, agg2, x)


def kernel(node_ids, edges, edge_types, embed_table, edge_embed_table,
           ggnn_weight, w_ih, w_hh, b_ih, b_hh, gate_w, gate_b):
    src = edges[0].astype(jnp.int32)
    dst = edges[1].astype(jnp.int32)
    et = edge_types.astype(jnp.int32)
    padi = jnp.arange(EP - E, dtype=jnp.int32) % 16
    src_p = jnp.concatenate([src, padi]).reshape(NW, EW)
    et_p = jnp.concatenate([et, jnp.ones((EP - E,), jnp.int32)]).reshape(NW, EW)
    dst_p = jnp.concatenate([dst, N + padi]).reshape(NW, NWIN, WSZ)
    ids_p = jnp.concatenate(
        [node_ids.astype(jnp.int32),
         jnp.arange(NP - N, dtype=jnp.int32) % 64]).reshape(NW, IDW)
    zrows = jnp.zeros((NP // NS, H), jnp.float32)
    bih = b_ih.reshape(1, 3 * H)
    bhh = b_hh.reshape(1, 3 * H)
    gb = gate_b.reshape(1, 1)

    x, s2 = _sc_embed_kernel()(embed_table, ids_p, src_p, et_p)
    s2 = s2.reshape(NW, NWIN, WSZ)
    m7 = _m7_call(edge_embed_table, ggnn_weight[0], x).reshape(7 * NP, H)
    for layer in range(L):
        agg2 = _sc_scatter_kernel()(m7, s2, dst_p, zrows)
        if layer + 1 < L:
            x, m7 = _ga_call(w_ih, w_hh, bih, bhh, edge_embed_table,
                             ggnn_weight[layer + 1], agg2, x)
            m7 = m7.reshape(7 * NP, H)
        else:
            return _gru_pool_call(w_ih, w_hh, bih, bhh, gate_w, gb, agg2, x)
